# baseline (speedup 1.0000x reference)
import functools

import jax
import jax.numpy as jnp
from jax import lax
from jax.experimental import pallas as pl
from jax.experimental.pallas import tpu as pltpu
from jax.experimental.pallas import tpu_sc as plsc

_V = 1000000
_D = 64
_OUT = 100
_B = 4096
_L = 200
_W = 20

_NC = 2
_NS = 16
_NW = _NC * _NS
_BPW = _B // _NW
_CHUNKS = ((0, 128), (128, 72))
_LANES = 16
_DV = _D // _LANES
_LP = 256

_CB = 2048


def _repack_kernel(tt_ref, out_ref):
    eye = jnp.eye(_D, dtype=jnp.float32)
    t = lax.dot_general(
        tt_ref[...], eye, (((0,), (0,)), ((), ())),
        preferred_element_type=jnp.float32)
    out_ref[:, 0:_D] = t
    out_ref[:, _D:2 * _D] = t


def _repack(tt):
    return pl.pallas_call(
        _repack_kernel,
        out_shape=jax.ShapeDtypeStruct((_V, 2 * _D), jnp.float32),
        grid=(pl.cdiv(_V, _CB),),
        in_specs=[pl.BlockSpec((_D, _CB), lambda i: (0, i))],
        out_specs=pl.BlockSpec((_CB, 2 * _D), lambda i: (i, 0)),
    )(tt)


def _sc_embed_sum(xf, rep):
    mesh = plsc.VectorSubcoreMesh(
        core_axis_name="c", subcore_axis_name="s",
        num_cores=_NC, num_subcores=_NS)

    @functools.partial(
        pl.kernel,
        out_type=jax.ShapeDtypeStruct((_B, _D), jnp.float32),
        mesh=mesh,
        compiler_params=pltpu.CompilerParams(use_tc_tiling_on_sc=False),
        scratch_types=[
            pltpu.VMEM((_BPW * _LP,), jnp.int32),
            pltpu.VMEM((2, _L, 2 * _D), jnp.float32),
            pltpu.VMEM((_BPW, _D), jnp.float32),
            pltpu.SemaphoreType.DMA,
            pltpu.SemaphoreType.DMA,
        ],
    )
    def body(x_hbm, tab_hbm, sent_hbm, idx_v, rows_v, sums_v, gsem0, gsem1):
        wid = lax.axis_index("s") * _NC + lax.axis_index("c")
        base = wid * _BPW
        pltpu.sync_copy(x_hbm.at[pl.ds(base * _LP, _BPW * _LP)], idx_v)
        sems = (gsem0, gsem1)

        def fire(i, slot):
            for off, sz in _CHUNKS:
                pltpu.async_copy(
                    tab_hbm.at[idx_v.at[pl.ds(i * _LP + off, sz)]],
                    rows_v.at[slot, pl.ds(off, sz)], sems[slot])

        def drain(slot):
            for off, sz in _CHUNKS:
                pltpu.make_async_copy(
                    tab_hbm.at[idx_v.at[pl.ds(off, sz)]],
                    rows_v.at[slot, pl.ds(off, sz)], sems[slot]).wait()

        fire(0, 0)
        fire(1, 1)

        @pl.loop(0, _BPW, step=2)
        def _rows(i):
            for b in range(2):
                ib = i + b
                drain(b)
                zero = jnp.zeros((_LANES,), jnp.float32)

                def red(c, carry):
                    return tuple(
                        carry[k] + rows_v[b, c, pl.ds(k * _LANES, _LANES)]
                        for k in range(_DV))

                acc = lax.fori_loop(0, _L, red, (zero,) * _DV, unroll=4)
                for k in range(_DV):
                    sums_v[ib, pl.ds(k * _LANES, _LANES)] = acc[k]

                @pl.when(ib + 2 < _BPW)
                def _():
                    fire(ib + 2, b)

        pltpu.sync_copy(sums_v, sent_hbm.at[pl.ds(base, _BPW)])

    return body(xf, rep)


def _fc_kernel(s_ref, w_ref, b_ref, o_ref):
    o_ref[...] = (
        jnp.dot(s_ref[...], w_ref[...], preferred_element_type=jnp.float32)
        * (1.0 / _W)
        + b_ref[...]
    )


def _fc(sent, w_t, fc_b):
    return pl.pallas_call(
        _fc_kernel,
        out_shape=jax.ShapeDtypeStruct((_B, _OUT), jnp.float32),
    )(sent, w_t, fc_b[None, :])


def kernel(x, word_incices, table, fc_w, fc_b):
    del word_incices
    rep = _repack(table.T)
    xf = jnp.pad(x, ((0, 0), (0, _LP - _L))).reshape(-1)
    sent = _sc_embed_sum(xf, rep)
    return _fc(sent, fc_w.T, fc_b)


# speedup vs baseline: 1.3589x; 1.3589x over previous
import functools

import jax
import jax.numpy as jnp
from jax import lax
from jax.experimental import pallas as pl
from jax.experimental.pallas import tpu as pltpu
from jax.experimental.pallas import tpu_sc as plsc

_V = 1000000
_D = 64
_OUT = 100
_B = 4096
_L = 200
_W = 20

_NC = 2
_NS = 16
_NW = _NC * _NS
_BPW = _B // _NW
_CHUNKS = ((0, 128), (128, 72))
_LANES = 16
_DV = _D // _LANES
_LP = 256

_CB = 8192


def _repack_kernel(tt_ref, out_ref):
    eye = jnp.eye(_D, dtype=jnp.float32)
    t = lax.dot_general(
        tt_ref[...], eye, (((0,), (0,)), ((), ())),
        preferred_element_type=jnp.float32)
    out_ref[:, 0:_D] = t
    out_ref[:, _D:2 * _D] = t


def _repack(tt):
    return pl.pallas_call(
        _repack_kernel,
        out_shape=jax.ShapeDtypeStruct((_V, 2 * _D), jnp.float32),
        grid=(pl.cdiv(_V, _CB),),
        in_specs=[pl.BlockSpec((_D, _CB), lambda i: (0, i))],
        out_specs=pl.BlockSpec((_CB, 2 * _D), lambda i: (i, 0)),
    )(tt)


def _sc_embed_sum(xf, rep):
    mesh = plsc.VectorSubcoreMesh(
        core_axis_name="c", subcore_axis_name="s",
        num_cores=_NC, num_subcores=_NS)

    @functools.partial(
        pl.kernel,
        out_type=jax.ShapeDtypeStruct((_B, _D), jnp.float32),
        mesh=mesh,
        compiler_params=pltpu.CompilerParams(use_tc_tiling_on_sc=False),
        scratch_types=[
            pltpu.VMEM((_BPW * _LP,), jnp.int32),
            pltpu.VMEM((2, _L, 2 * _D), jnp.float32),
            pltpu.VMEM((_BPW, _D), jnp.float32),
            pltpu.SemaphoreType.DMA,
            pltpu.SemaphoreType.DMA,
        ],
    )
    def body(x_hbm, tab_hbm, sent_hbm, idx_v, rows_v, sums_v, gsem0, gsem1):
        wid = lax.axis_index("s") * _NC + lax.axis_index("c")
        base = wid * _BPW
        pltpu.sync_copy(x_hbm.at[pl.ds(base * _LP, _BPW * _LP)], idx_v)
        sems = (gsem0, gsem1)

        def fire(i, slot):
            for off, sz in _CHUNKS:
                pltpu.async_copy(
                    tab_hbm.at[idx_v.at[pl.ds(i * _LP + off, sz)]],
                    rows_v.at[slot, pl.ds(off, sz)], sems[slot])

        def drain(slot):
            for off, sz in _CHUNKS:
                pltpu.make_async_copy(
                    tab_hbm.at[idx_v.at[pl.ds(off, sz)]],
                    rows_v.at[slot, pl.ds(off, sz)], sems[slot]).wait()

        fire(0, 0)
        fire(1, 1)

        @pl.loop(0, _BPW, step=2)
        def _rows(i):
            for b in range(2):
                ib = i + b
                drain(b)
                zero = jnp.zeros((_LANES,), jnp.float32)

                def red(c, carry):
                    return tuple(
                        carry[k] + rows_v[b, c, pl.ds(k * _LANES, _LANES)]
                        for k in range(_DV))

                acc = lax.fori_loop(0, _L, red, (zero,) * _DV, unroll=4)
                for k in range(_DV):
                    sums_v[ib, pl.ds(k * _LANES, _LANES)] = acc[k]

                @pl.when(ib + 2 < _BPW)
                def _():
                    fire(ib + 2, b)

        pltpu.sync_copy(sums_v, sent_hbm.at[pl.ds(base, _BPW)])

    return body(xf, rep)


def _fc_kernel(s_ref, w_ref, b_ref, o_ref):
    o_ref[...] = (
        jnp.dot(s_ref[...], w_ref[...], preferred_element_type=jnp.float32)
        * (1.0 / _W)
        + b_ref[...]
    )


def _fc(sent, w_t, fc_b):
    return pl.pallas_call(
        _fc_kernel,
        out_shape=jax.ShapeDtypeStruct((_B, _OUT), jnp.float32),
    )(sent, w_t, fc_b[None, :])


def kernel(x, word_incices, table, fc_w, fc_b):
    del word_incices
    rep = _repack(table.T)
    xf = jnp.pad(x, ((0, 0), (0, _LP - _L))).reshape(-1)
    sent = _sc_embed_sum(xf, rep)
    return _fc(sent, fc_w.T, fc_b)


# speedup vs baseline: 1.4504x; 1.0673x over previous
import functools

import jax
import jax.numpy as jnp
from jax import lax
from jax.experimental import pallas as pl
from jax.experimental.pallas import tpu as pltpu
from jax.experimental.pallas import tpu_sc as plsc

_V = 1000000
_D = 64
_OUT = 100
_B = 4096
_L = 200
_W = 20

_NC = 2
_NS = 16
_NW = _NC * _NS
_BPW = _B // _NW
_CHUNKS = ((0, 128), (128, 72))
_LANES = 16
_DV = _D // _LANES
_LP = 256

_CB = 16384


def _repack_kernel(tt_ref, out_ref):
    eye = jnp.eye(_D, dtype=jnp.float32)
    t = lax.dot_general(
        tt_ref[...], eye, (((0,), (0,)), ((), ())),
        preferred_element_type=jnp.float32)
    out_ref[:, 0:_D] = t
    out_ref[:, _D:2 * _D] = t


def _repack(tt):
    return pl.pallas_call(
        _repack_kernel,
        out_shape=jax.ShapeDtypeStruct((_V, 2 * _D), jnp.float32),
        grid=(pl.cdiv(_V, _CB),),
        in_specs=[pl.BlockSpec((_D, _CB), lambda i: (0, i))],
        out_specs=pl.BlockSpec((_CB, 2 * _D), lambda i: (i, 0)),
    )(tt)


def _sc_embed_sum(xf, rep):
    mesh = plsc.VectorSubcoreMesh(
        core_axis_name="c", subcore_axis_name="s",
        num_cores=_NC, num_subcores=_NS)

    @functools.partial(
        pl.kernel,
        out_type=jax.ShapeDtypeStruct((_B, _D), jnp.float32),
        mesh=mesh,
        compiler_params=pltpu.CompilerParams(use_tc_tiling_on_sc=False),
        scratch_types=[
            pltpu.VMEM((_BPW * _LP,), jnp.int32),
            pltpu.VMEM((2, _L, 2 * _D), jnp.float32),
            pltpu.VMEM((_BPW, _D), jnp.float32),
            pltpu.SemaphoreType.DMA,
            pltpu.SemaphoreType.DMA,
        ],
    )
    def body(x_hbm, tab_hbm, sent_hbm, idx_v, rows_v, sums_v, gsem0, gsem1):
        wid = lax.axis_index("s") * _NC + lax.axis_index("c")
        base = wid * _BPW
        pltpu.sync_copy(x_hbm.at[pl.ds(base * _LP, _BPW * _LP)], idx_v)
        sems = (gsem0, gsem1)

        def fire(i, slot):
            for off, sz in _CHUNKS:
                pltpu.async_copy(
                    tab_hbm.at[idx_v.at[pl.ds(i * _LP + off, sz)]],
                    rows_v.at[slot, pl.ds(off, sz)], sems[slot])

        def drain(slot):
            for off, sz in _CHUNKS:
                pltpu.make_async_copy(
                    tab_hbm.at[idx_v.at[pl.ds(off, sz)]],
                    rows_v.at[slot, pl.ds(off, sz)], sems[slot]).wait()

        fire(0, 0)
        fire(1, 1)

        @pl.loop(0, _BPW, step=2)
        def _rows(i):
            for b in range(2):
                ib = i + b
                drain(b)
                zero = jnp.zeros((_LANES,), jnp.float32)

                def red(c, carry):
                    return tuple(
                        carry[k] + rows_v[b, c, pl.ds(k * _LANES, _LANES)]
                        for k in range(_DV))

                acc = lax.fori_loop(0, _L, red, (zero,) * _DV, unroll=4)
                for k in range(_DV):
                    sums_v[ib, pl.ds(k * _LANES, _LANES)] = acc[k]

                @pl.when(ib + 2 < _BPW)
                def _():
                    fire(ib + 2, b)

        pltpu.sync_copy(sums_v, sent_hbm.at[pl.ds(base, _BPW)])

    return body(xf, rep)


def _fc_kernel(s_ref, w_ref, b_ref, o_ref):
    o_ref[...] = (
        jnp.dot(s_ref[...], w_ref[...], preferred_element_type=jnp.float32)
        * (1.0 / _W)
        + b_ref[...]
    )


def _fc(sent, w_t, fc_b):
    return pl.pallas_call(
        _fc_kernel,
        out_shape=jax.ShapeDtypeStruct((_B, _OUT), jnp.float32),
    )(sent, w_t, fc_b[None, :])


def kernel(x, word_incices, table, fc_w, fc_b):
    del word_incices
    rep = _repack(table.T)
    xf = jnp.pad(x, ((0, 0), (0, _LP - _L))).reshape(-1)
    sent = _sc_embed_sum(xf, rep)
    return _fc(sent, fc_w.T, fc_b)


# speedup vs baseline: 1.5162x; 1.0454x over previous
import functools

import jax
import jax.numpy as jnp
from jax import lax
from jax.experimental import pallas as pl
from jax.experimental.pallas import tpu as pltpu
from jax.experimental.pallas import tpu_sc as plsc

_V = 1000000
_D = 64
_OUT = 100
_B = 4096
_L = 200
_W = 20

_NC = 2
_NS = 16
_NW = _NC * _NS
_BPW = _B // _NW
_CHUNKS = ((0, 128), (128, 72))
_LANES = 16
_DV = _D // _LANES
_LP = 256

_CB = 16384


def _repack_kernel(tt_ref, out_ref):
    eye = jnp.eye(_D, dtype=jnp.float32)
    t = lax.dot_general(
        tt_ref[...], eye, (((0,), (0,)), ((), ())),
        preferred_element_type=jnp.float32)
    out_ref[:, 0:_D] = t
    out_ref[:, _D:2 * _D] = t


def _repack(tt):
    return pl.pallas_call(
        _repack_kernel,
        out_shape=jax.ShapeDtypeStruct((_V, 2 * _D), jnp.float32),
        grid=(pl.cdiv(_V, _CB),),
        in_specs=[pl.BlockSpec((_D, _CB), lambda i: (0, i))],
        out_specs=pl.BlockSpec((_CB, 2 * _D), lambda i: (i, 0)),
    )(tt)


def _sc_embed_sum(xf, rep):
    mesh = plsc.VectorSubcoreMesh(
        core_axis_name="c", subcore_axis_name="s",
        num_cores=_NC, num_subcores=_NS)

    @functools.partial(
        pl.kernel,
        out_type=jax.ShapeDtypeStruct((_B, _D), jnp.float32),
        mesh=mesh,
        compiler_params=pltpu.CompilerParams(use_tc_tiling_on_sc=False),
        scratch_types=[
            pltpu.VMEM((_BPW * _LP,), jnp.int32),
            pltpu.VMEM((3, _L, 2 * _D), jnp.float32),
            pltpu.VMEM((_BPW, _D), jnp.float32),
            pltpu.SemaphoreType.DMA,
            pltpu.SemaphoreType.DMA,
            pltpu.SemaphoreType.DMA,
        ],
    )
    def body(x_hbm, tab_hbm, sent_hbm, idx_v, rows_v, sums_v,
             gsem0, gsem1, gsem2):
        wid = lax.axis_index("s") * _NC + lax.axis_index("c")
        base = wid * _BPW
        pltpu.sync_copy(x_hbm.at[pl.ds(base * _LP, _BPW * _LP)], idx_v)
        sems = (gsem0, gsem1, gsem2)

        def fire(i, slot):
            for off, sz in _CHUNKS:
                pltpu.async_copy(
                    tab_hbm.at[idx_v.at[pl.ds(i * _LP + off, sz)]],
                    rows_v.at[slot, pl.ds(off, sz)], sems[slot])

        def drain(slot):
            for off, sz in _CHUNKS:
                pltpu.make_async_copy(
                    tab_hbm.at[idx_v.at[pl.ds(off, sz)]],
                    rows_v.at[slot, pl.ds(off, sz)], sems[slot]).wait()

        fire(0, 0)
        fire(1, 1)
        fire(2, 2)

        @pl.loop(0, _BPW + 1, step=3)
        def _rows(i):
            for b in range(3):
                ib = i + b

                @pl.when(ib < _BPW)
                def _():
                    drain(b)
                    zero = jnp.zeros((_LANES,), jnp.float32)

                    def red(c, carry):
                        return tuple(
                            carry[k] + rows_v[b, c, pl.ds(k * _LANES, _LANES)]
                            for k in range(_DV))

                    acc = lax.fori_loop(0, _L, red, (zero,) * _DV, unroll=4)
                    for k in range(_DV):
                        sums_v[ib, pl.ds(k * _LANES, _LANES)] = acc[k]

                    @pl.when(ib + 3 < _BPW)
                    def _():
                        fire(ib + 3, b)

        pltpu.sync_copy(sums_v, sent_hbm.at[pl.ds(base, _BPW)])

    return body(xf, rep)


def _fc_kernel(s_ref, w_ref, b_ref, o_ref):
    o_ref[...] = (
        jnp.dot(s_ref[...], w_ref[...], preferred_element_type=jnp.float32)
        * (1.0 / _W)
        + b_ref[...]
    )


def _fc(sent, w_t, fc_b):
    return pl.pallas_call(
        _fc_kernel,
        out_shape=jax.ShapeDtypeStruct((_B, _OUT), jnp.float32),
    )(sent, w_t, fc_b[None, :])


def kernel(x, word_incices, table, fc_w, fc_b):
    del word_incices
    rep = _repack(table.T)
    xf = jnp.pad(x, ((0, 0), (0, _LP - _L))).reshape(-1)
    sent = _sc_embed_sum(xf, rep)
    return _fc(sent, fc_w.T, fc_b)


# speedup vs baseline: 1.5462x; 1.0198x over previous
import functools

import jax
import jax.numpy as jnp
from jax import lax
from jax.experimental import pallas as pl
from jax.experimental.pallas import tpu as pltpu
from jax.experimental.pallas import tpu_sc as plsc

_V = 1000000
_D = 64
_OUT = 100
_B = 4096
_L = 200
_W = 20

_NC = 2
_NS = 16
_NW = _NC * _NS
_BPW = _B // _NW
_CHUNKS = ((0, 128), (128, 72))
_LANES = 16
_DV = _D // _LANES
_LP = 256

_CB = 24576


def _repack_kernel(tt_ref, out_ref):
    eye = jnp.eye(_D, dtype=jnp.float32)
    t = lax.dot_general(
        tt_ref[...], eye, (((0,), (0,)), ((), ())),
        preferred_element_type=jnp.float32)
    out_ref[:, 0:_D] = t
    out_ref[:, _D:2 * _D] = t


def _repack(tt):
    return pl.pallas_call(
        _repack_kernel,
        out_shape=jax.ShapeDtypeStruct((_V, 2 * _D), jnp.float32),
        grid=(pl.cdiv(_V, _CB),),
        in_specs=[pl.BlockSpec((_D, _CB), lambda i: (0, i))],
        out_specs=pl.BlockSpec((_CB, 2 * _D), lambda i: (i, 0)),
    )(tt)


def _sc_embed_sum(xf, rep):
    mesh = plsc.VectorSubcoreMesh(
        core_axis_name="c", subcore_axis_name="s",
        num_cores=_NC, num_subcores=_NS)

    @functools.partial(
        pl.kernel,
        out_type=jax.ShapeDtypeStruct((_B, _D), jnp.float32),
        mesh=mesh,
        compiler_params=pltpu.CompilerParams(use_tc_tiling_on_sc=False),
        scratch_types=[
            pltpu.VMEM((_BPW * _LP,), jnp.int32),
            pltpu.VMEM((3, _L, 2 * _D), jnp.float32),
            pltpu.VMEM((_BPW, _D), jnp.float32),
            pltpu.SemaphoreType.DMA,
            pltpu.SemaphoreType.DMA,
            pltpu.SemaphoreType.DMA,
        ],
    )
    def body(x_hbm, tab_hbm, sent_hbm, idx_v, rows_v, sums_v,
             gsem0, gsem1, gsem2):
        wid = lax.axis_index("s") * _NC + lax.axis_index("c")
        base = wid * _BPW
        pltpu.sync_copy(x_hbm.at[pl.ds(base * _LP, _BPW * _LP)], idx_v)
        sems = (gsem0, gsem1, gsem2)

        def fire(i, slot):
            for off, sz in _CHUNKS:
                pltpu.async_copy(
                    tab_hbm.at[idx_v.at[pl.ds(i * _LP + off, sz)]],
                    rows_v.at[slot, pl.ds(off, sz)], sems[slot])

        def drain(slot):
            for off, sz in _CHUNKS:
                pltpu.make_async_copy(
                    tab_hbm.at[idx_v.at[pl.ds(off, sz)]],
                    rows_v.at[slot, pl.ds(off, sz)], sems[slot]).wait()

        fire(0, 0)
        fire(1, 1)
        fire(2, 2)

        @pl.loop(0, _BPW + 1, step=3)
        def _rows(i):
            for b in range(3):
                ib = i + b

                @pl.when(ib < _BPW)
                def _():
                    drain(b)
                    zero = jnp.zeros((_LANES,), jnp.float32)

                    def red(c, carry):
                        return tuple(
                            carry[k] + rows_v[b, c, pl.ds(k * _LANES, _LANES)]
                            for k in range(_DV))

                    acc = lax.fori_loop(0, _L, red, (zero,) * _DV, unroll=8)
                    for k in range(_DV):
                        sums_v[ib, pl.ds(k * _LANES, _LANES)] = acc[k]

                    @pl.when(ib + 3 < _BPW)
                    def _():
                        fire(ib + 3, b)

        pltpu.sync_copy(sums_v, sent_hbm.at[pl.ds(base, _BPW)])

    return body(xf, rep)


def _fc_kernel(s_ref, w_ref, b_ref, o_ref):
    o_ref[...] = (
        jnp.dot(s_ref[...], w_ref[...], preferred_element_type=jnp.float32)
        * (1.0 / _W)
        + b_ref[...]
    )


def _fc(sent, w_t, fc_b):
    return pl.pallas_call(
        _fc_kernel,
        out_shape=jax.ShapeDtypeStruct((_B, _OUT), jnp.float32),
    )(sent, w_t, fc_b[None, :])


def kernel(x, word_incices, table, fc_w, fc_b):
    del word_incices
    rep = _repack(table.T)
    xf = jnp.pad(x, ((0, 0), (0, _LP - _L))).reshape(-1)
    sent = _sc_embed_sum(xf, rep)
    return _fc(sent, fc_w.T, fc_b)
